# baseline (speedup 1.0000x reference)
import jax
import jax.numpy as jnp
from jax import lax
from jax.experimental import pallas as pl
from jax.experimental.pallas import tpu as pltpu
from jax.experimental.pallas import tpu_sc as plsc

NUM_CORES = 2
NUM_SUBCORES = 16
LANES = 16
NW = NUM_CORES * NUM_SUBCORES

UROWS = 1000000 // LANES
MROWS = 100000 // LANES

BATCH = 16384
EMBED_DIM = 64
B_PER_W = BATCH // NW
CHUNK = 128
NCHUNK = B_PER_W // CHUNK
GPC = CHUNK // LANES
NBUF = 4


def _sc_kernel(user_input, movie_input, ue16, me16, ub16, mb16, out_hbm,
               idx_u, idx_m, bidx_u, bidx_m, colu_v, colm_v,
               rows_u, rows_m, bias_u, bias_m, out_v, sem_i, *sems):
    wid = lax.axis_index("s") * NUM_CORES + lax.axis_index("c")
    base = wid * B_PER_W

    icopies = []
    for j in range(NCHUNK):
        off = base + j * CHUNK
        icopies.append(pltpu.async_copy(user_input.at[pl.ds(off, CHUNK)], idx_u.at[j], sem_i))
        icopies.append(pltpu.async_copy(movie_input.at[pl.ds(off, CHUNK)], idx_m.at[j], sem_i))
    for c in icopies:
        c.wait()
    for j in range(NCHUNK):
        for k in range(GPC):
            s = pl.ds(k * LANES, LANES)
            iu = idx_u[j, s]
            im = idx_m[j, s]
            bidx_u[j, s] = iu >> 4
            bidx_m[j, s] = im >> 4
            colu_v[j, s] = iu & 15
            colm_v[j, s] = im & 15

    bcopies = []
    for j in range(NCHUNK):
        bcopies.append(pltpu.async_copy(ub16.at[bidx_u.at[j]], bias_u.at[j], sem_i))
        bcopies.append(pltpu.async_copy(mb16.at[bidx_m.at[j]], bias_m.at[j], sem_i))
    for c in bcopies:
        c.wait()

    lane = lax.iota(jnp.int32, LANES)

    def fire(j, d, b):
        pltpu.async_copy(
            ue16.at[pl.ds(d * UROWS, UROWS)].at[bidx_u.at[j]],
            rows_u.at[b], sems[b])
        pltpu.async_copy(
            me16.at[pl.ds(d * MROWS, MROWS)].at[bidx_m.at[j]],
            rows_m.at[b], sems[b])

    def drain(j, b):
        pltpu.make_async_copy(
            ue16.at[pl.ds(0, UROWS)].at[bidx_u.at[j]],
            rows_u.at[b], sems[b]).wait()
        pltpu.make_async_copy(
            me16.at[pl.ds(0, MROWS)].at[bidx_m.at[j]],
            rows_m.at[b], sems[b]).wait()

    for j in range(NCHUNK):
        jvec = jnp.full((LANES,), j, jnp.int32)
        for b in range(NBUF):
            fire(j, b, b)

        def block(blk, accs):
            accs = list(accs)
            d0 = blk * NBUF
            for b in range(NBUF):
                drain(j, b)
                bvec = jnp.full((LANES,), b, jnp.int32)
                for k in range(GPC):
                    rows = k * LANES + lane
                    cu = plsc.load_gather(colu_v, [jvec, rows])
                    cm = plsc.load_gather(colm_v, [jvec, rows])
                    u = plsc.load_gather(rows_u, [bvec, rows, cu])
                    m = plsc.load_gather(rows_m, [bvec, rows, cm])
                    accs[k] = accs[k] + u * m
                nd = d0 + b + NBUF

                @pl.when(nd < EMBED_DIM)
                def _():
                    fire(j, nd, b)

            return tuple(accs)

        accs = lax.fori_loop(
            0, EMBED_DIM // NBUF, block,
            tuple(jnp.zeros((LANES,), jnp.float32) for _ in range(GPC)))

        for k in range(GPC):
            rows = k * LANES + lane
            iu = plsc.load_gather(colu_v, [jvec, rows])
            im = plsc.load_gather(colm_v, [jvec, rows])
            bu = plsc.load_gather(bias_u, [jvec, rows, iu])
            bm = plsc.load_gather(bias_m, [jvec, rows, im])
            x = accs[k] + bu + bm
            y = 1.0 / (1.0 + jnp.exp(-x))
            plsc.store_scatter(out_v, [j * CHUNK + rows], y)

    pltpu.sync_copy(out_v, out_hbm.at[pl.ds(base, B_PER_W)])


def kernel(user_input, movie_input, user_emb, user_bias, movie_emb, movie_bias):
    mesh = plsc.VectorSubcoreMesh(
        core_axis_name="c", subcore_axis_name="s",
        num_cores=NUM_CORES, num_subcores=NUM_SUBCORES)
    f = pl.kernel(
        _sc_kernel,
        mesh=mesh,
        compiler_params=pltpu.CompilerParams(
            needs_layout_passes=False, use_tc_tiling_on_sc=False),
        out_type=jax.ShapeDtypeStruct((BATCH,), jnp.float32),
        scratch_types=[
            pltpu.VMEM((NCHUNK, CHUNK), jnp.int32),
            pltpu.VMEM((NCHUNK, CHUNK), jnp.int32),
            pltpu.VMEM((NCHUNK, CHUNK), jnp.int32),
            pltpu.VMEM((NCHUNK, CHUNK), jnp.int32),
            pltpu.VMEM((NCHUNK, CHUNK), jnp.int32),
            pltpu.VMEM((NCHUNK, CHUNK), jnp.int32),
            pltpu.VMEM((NBUF, CHUNK, LANES), jnp.float32),
            pltpu.VMEM((NBUF, CHUNK, LANES), jnp.float32),
            pltpu.VMEM((NCHUNK, CHUNK, LANES), jnp.float32),
            pltpu.VMEM((NCHUNK, CHUNK, LANES), jnp.float32),
            pltpu.VMEM((B_PER_W,), jnp.float32),
            pltpu.SemaphoreType.DMA,
        ] + [pltpu.SemaphoreType.DMA] * NBUF,
    )
    ue16 = user_emb.T.reshape(EMBED_DIM * UROWS, LANES)
    me16 = movie_emb.T.reshape(EMBED_DIM * MROWS, LANES)
    ub16 = user_bias.reshape(user_bias.shape[0] // LANES, LANES)
    mb16 = movie_bias.reshape(movie_bias.shape[0] // LANES, LANES)
    return f(user_input, movie_input, ue16, me16, ub16, mb16)


# speedup vs baseline: 7.5291x; 7.5291x over previous
import jax
import jax.numpy as jnp
from jax import lax
from jax.experimental import pallas as pl
from jax.experimental.pallas import tpu as pltpu
from jax.experimental.pallas import tpu_sc as plsc

NUM_CORES = 2
NUM_SUBCORES = 16
LANES = 16
NW = NUM_CORES * NUM_SUBCORES

BATCH = 16384
EMBED_DIM = 64
B_PER_W = BATCH // NW
CHUNK = 128
NCHUNK = B_PER_W // CHUNK
GROUPS = B_PER_W // LANES


def _sc_kernel(user_input, movie_input, user_emb, user_bias, movie_emb,
               movie_bias, out_hbm, idx_u, idx_m, bidx_u, bidx_m,
               rows_u, rows_m, bias_u, bias_m, out_v, sem):
    wid = lax.axis_index("s") * NUM_CORES + lax.axis_index("c")
    base = wid * B_PER_W

    icopies = []
    for j in range(NCHUNK):
        off = base + j * CHUNK
        icopies.append(pltpu.async_copy(user_input.at[pl.ds(off, CHUNK)], idx_u.at[j], sem))
        icopies.append(pltpu.async_copy(movie_input.at[pl.ds(off, CHUNK)], idx_m.at[j], sem))
    for c in icopies:
        c.wait()
    for j in range(NCHUNK):
        for k in range(CHUNK // LANES):
            s = pl.ds(k * LANES, LANES)
            bidx_u[j, s] = idx_u[j, s] >> 4
            bidx_m[j, s] = idx_m[j, s] >> 4

    copies = []
    for j in range(NCHUNK):
        copies.append(pltpu.async_copy(user_emb.at[idx_u.at[j]], rows_u.at[j], sem))
        copies.append(pltpu.async_copy(movie_emb.at[idx_m.at[j]], rows_m.at[j], sem))
        copies.append(pltpu.async_copy(user_bias.at[bidx_u.at[j]], bias_u.at[j], sem))
        copies.append(pltpu.async_copy(movie_bias.at[bidx_m.at[j]], bias_m.at[j], sem))
    for c in copies:
        c.wait()

    lane = lax.iota(jnp.int32, LANES)

    def group_body(g, carry):
        j = g // (CHUNK // LANES)
        r = (g % (CHUNK // LANES)) * LANES
        jvec = jnp.full((LANES,), j, jnp.int32)
        rows = r + lane
        acc = jnp.zeros((LANES,), jnp.float32)
        for d in range(EMBED_DIM):
            dvec = jnp.full((LANES,), d, jnp.int32)
            u = plsc.load_gather(rows_u, [jvec, rows, dvec])
            m = plsc.load_gather(rows_m, [jvec, rows, dvec])
            acc = acc + u * m
        iu = plsc.load_gather(idx_u, [jvec, rows])
        im = plsc.load_gather(idx_m, [jvec, rows])
        bu = plsc.load_gather(bias_u, [jvec, rows, iu & 15])
        bm = plsc.load_gather(bias_m, [jvec, rows, im & 15])
        x = acc + bu + bm
        y = 1.0 / (1.0 + jnp.exp(-x))
        plsc.store_scatter(out_v, [g * LANES + lane], y)
        return carry

    lax.fori_loop(0, GROUPS, group_body, 0)

    pltpu.sync_copy(out_v, out_hbm.at[pl.ds(base, B_PER_W)])


def kernel(user_input, movie_input, user_emb, user_bias, movie_emb, movie_bias):
    mesh = plsc.VectorSubcoreMesh(
        core_axis_name="c", subcore_axis_name="s",
        num_cores=NUM_CORES, num_subcores=NUM_SUBCORES)
    f = pl.kernel(
        _sc_kernel,
        mesh=mesh,
        compiler_params=pltpu.CompilerParams(
            needs_layout_passes=False, use_tc_tiling_on_sc=False),
        out_type=jax.ShapeDtypeStruct((BATCH,), jnp.float32),
        scratch_types=[
            pltpu.VMEM((NCHUNK, CHUNK), jnp.int32),
            pltpu.VMEM((NCHUNK, CHUNK), jnp.int32),
            pltpu.VMEM((NCHUNK, CHUNK), jnp.int32),
            pltpu.VMEM((NCHUNK, CHUNK), jnp.int32),
            pltpu.VMEM((NCHUNK, CHUNK, EMBED_DIM), jnp.float32),
            pltpu.VMEM((NCHUNK, CHUNK, EMBED_DIM), jnp.float32),
            pltpu.VMEM((NCHUNK, CHUNK, LANES), jnp.float32),
            pltpu.VMEM((NCHUNK, CHUNK, LANES), jnp.float32),
            pltpu.VMEM((B_PER_W,), jnp.float32),
            pltpu.SemaphoreType.DMA,
        ],
    )
    ub16 = user_bias.reshape(user_bias.shape[0] // LANES, LANES)
    mb16 = movie_bias.reshape(movie_bias.shape[0] // LANES, LANES)
    return f(user_input, movie_input, user_emb, ub16, movie_emb, mb16)
